# baseline (speedup 1.0000x reference)
import functools

import jax
import jax.numpy as jnp
from jax import lax
from jax.experimental import pallas as pl
from jax.experimental.pallas import tpu as pltpu
from jax.experimental.pallas import tpu_sc as plsc

ALPHA = 0.3
BETA = 1.0 - ALPHA
N = 10000
E = 320000
G = 625
IN_CH = 128
HID = 172
HIDP = 192
CH = HIDP // 2
OUT_CH = 10

BB = 20
NBT = E // (16 * BB)
RCHUNK = 632


def _sc_mesh():
    return plsc.VectorSubcoreMesh(core_axis_name="c", subcore_axis_name="s")


def _part_copy(sid, src, dst):
    r0 = pl.multiple_of(sid * RCHUNK, 8)
    tail = N - 15 * RCHUNK

    @pl.when(sid < 15)
    def _():
        pltpu.sync_copy(src.at[pl.ds(r0, RCHUNK)], dst.at[pl.ds(r0, RCHUNK)])

    @pl.when(sid == 15)
    def _():
        pltpu.sync_copy(src.at[pl.ds(15 * RCHUNK, tail)],
                        dst.at[pl.ds(15 * RCHUNK, tail)])


def _deg_call(rc4, ewsp, zeros128):
    @functools.partial(
        pl.kernel,
        mesh=_sc_mesh(),
        out_type=jax.ShapeDtypeStruct((2, N, IN_CH), jnp.float32),
        scratch_types=[
            pltpu.VMEM((2, BB), jnp.int32),
            pltpu.VMEM((2, BB), jnp.int32),
            pltpu.VMEM((BB, 16), jnp.float32),
            pltpu.VMEM((BB, 16), jnp.float32),
            pltpu.VMEM((BB, IN_CH), jnp.float32),
            pltpu.VMEM((BB, IN_CH), jnp.float32),
            pltpu.VMEM_SHARED((N, IN_CH), jnp.float32),
            pltpu.SemaphoreType.DMA,
            pltpu.SemaphoreType.DMA,
            pltpu.SemaphoreType.DMA,
            pltpu.SemaphoreType.DMA,
            pltpu.SemaphoreType.DMA,
            pltpu.SemaphoreType.DMA,
        ],
    )
    def k(rc_hbm, ewsp_hbm, z_hbm, out_hbm,
          rcA, rcB, wspA, wspB, stA, stB, acc, e0, e1, w0, w1, s0, s1):
        cid = lax.axis_index("c")
        sid = lax.axis_index("s")
        nj = NBT // 2

        def zb(b, _):
            for kk in range(IN_CH // 16):
                stA[b, pl.ds(kk * 16, 16)] = jnp.zeros((16,), jnp.float32)
                stB[b, pl.ds(kk * 16, 16)] = jnp.zeros((16,), jnp.float32)
            return 0

        lax.fori_loop(0, BB, zb, 0)
        _part_copy(sid, z_hbm, acc)
        plsc.subcore_barrier()

        def body(jj, _):
            j0 = jj * 2
            jb0 = cid * nj + j0
            jb1 = jb0 + 1
            hr0 = pltpu.make_async_copy(rc_hbm.at[sid, jb0], rcA, e0)
            hw0 = pltpu.make_async_copy(ewsp_hbm.at[sid * NBT + jb0],
                                        wspA, w0)
            hr1 = pltpu.make_async_copy(rc_hbm.at[sid, jb1], rcB, e1)
            hw1 = pltpu.make_async_copy(ewsp_hbm.at[sid * NBT + jb1],
                                        wspB, w1)
            hr0.start()
            hw0.start()
            hr1.start()
            hw1.start()
            hr0.wait()
            hw0.wait()

            def cbA(b, _):
                stA[b, pl.ds(0, 16)] = wspA[b, :]
                return 0

            lax.fori_loop(0, BB, cbA, 0)
            s0c = pltpu.make_async_copy(stA, acc.at[rcA.at[1]], s0)
            s0c.start(add=True)
            hr1.wait()
            hw1.wait()

            def cbB(b, _):
                stB[b, pl.ds(0, 16)] = wspB[b, :]
                return 0

            lax.fori_loop(0, BB, cbB, 0)
            s1c = pltpu.make_async_copy(stB, acc.at[rcB.at[1]], s1)
            s1c.start(add=True)
            s0c.wait()
            s1c.wait()
            return 0

        lax.fori_loop(0, nj // 2, body, 0)
        plsc.subcore_barrier()
        _part_copy(sid, acc, out_hbm.at[cid])

    return k(rc4, ewsp, zeros128)


def _prop_call(rc4, ewsp, ta, tb, edge_split):
    nj = NBT // 2 if edge_split else NBT

    @functools.partial(
        pl.kernel,
        mesh=_sc_mesh(),
        out_type=[jax.ShapeDtypeStruct((N, IN_CH), jnp.float32)] * 2,
        scratch_types=[
            pltpu.VMEM((2, BB), jnp.int32),
            pltpu.VMEM((2, BB), jnp.int32),
            pltpu.VMEM((2, BB), jnp.int32),
            pltpu.VMEM((2, BB), jnp.int32),
            pltpu.VMEM((BB, 16), jnp.float32),
            pltpu.VMEM((BB, 16), jnp.float32),
            pltpu.VMEM((BB, IN_CH), jnp.float32),
            pltpu.VMEM((BB, IN_CH), jnp.float32),
            pltpu.VMEM_SHARED((N, IN_CH), jnp.float32),
        ] + [pltpu.SemaphoreType.DMA] * 10,
    )
    def k(rc_hbm, ewsp_hbm, ua_hbm, ub_hbm, outa, outb,
          rcA, rcB, rcC, rcD, wspA, wspB, rowsA, rowsB, acc,
          erA, erB, erC, erD, ewA, ewB, g0, g1, s0, s1):
        cid = lax.axis_index("c")
        sid = lax.axis_index("s")

        def run(init_hbm, u_hbm, out_hbm):
            _part_copy(sid, init_hbm, acc)
            plsc.subcore_barrier()

            def jb_of(j):
                return cid * nj + j if edge_split else j

            def scale(rows, wsp):
                def ebody(q, _):
                    for u in range(4):
                        b = q * 4 + u
                        w = wsp[b, :]
                        for kk in range(IN_CH // 16):
                            sl2 = pl.ds(kk * 16, 16)
                            rows[b, sl2] = rows[b, sl2] * w
                    return 0

                lax.fori_loop(0, BB // 4, ebody, 0)

            mac = pltpu.make_async_copy

            def body(jj, _):
                j = jj * 4
                jb0, jb1, jb2, jb3 = (jb_of(j), jb_of(j + 1),
                                      jb_of(j + 2), jb_of(j + 3))
                hrA = mac(rc_hbm.at[sid, jb0], rcA, erA)
                hrB = mac(rc_hbm.at[sid, jb1], rcB, erB)
                hrC = mac(rc_hbm.at[sid, jb2], rcC, erC)
                hrD = mac(rc_hbm.at[sid, jb3], rcD, erD)
                hwA = mac(ewsp_hbm.at[sid * NBT + jb0], wspA, ewA)
                hwB = mac(ewsp_hbm.at[sid * NBT + jb1], wspB, ewB)
                hrA.start()
                hrB.start()
                hrC.start()
                hrD.start()
                hwA.start()
                hwB.start()
                hrA.wait()
                gA = mac(u_hbm.at[rcA.at[0]], rowsA, g0)
                gA.start()
                hrB.wait()
                gB = mac(u_hbm.at[rcB.at[0]], rowsB, g1)
                gB.start()
                gA.wait()
                hwA.wait()
                scale(rowsA, wspA)
                sA = mac(rowsA, acc.at[rcA.at[1]], s0)
                sA.start(add=True)
                hwA2 = mac(ewsp_hbm.at[sid * NBT + jb2], wspA, ewA)
                hwA2.start()
                gB.wait()
                hwB.wait()
                scale(rowsB, wspB)
                sB = mac(rowsB, acc.at[rcB.at[1]], s1)
                sB.start(add=True)
                hwB2 = mac(ewsp_hbm.at[sid * NBT + jb3], wspB, ewB)
                hwB2.start()
                sA.wait()
                hrC.wait()
                gC = mac(u_hbm.at[rcC.at[0]], rowsA, g0)
                gC.start()
                sB.wait()
                hrD.wait()
                gD = mac(u_hbm.at[rcD.at[0]], rowsB, g1)
                gD.start()
                gC.wait()
                hwA2.wait()
                scale(rowsA, wspA)
                sC = mac(rowsA, acc.at[rcC.at[1]], s0)
                sC.start(add=True)
                gD.wait()
                hwB2.wait()
                scale(rowsB, wspB)
                sD = mac(rowsB, acc.at[rcD.at[1]], s1)
                sD.start(add=True)
                sC.wait()
                sD.wait()
                return 0

            lax.fori_loop(0, nj // 4, body, 0)
            plsc.subcore_barrier()
            _part_copy(sid, acc, out_hbm)
            plsc.subcore_barrier()

        @pl.when(cid == 0)
        def _():
            run(ua_hbm, ua_hbm, outa)

        @pl.when(cid != 0)
        def _():
            if edge_split:
                run(ub_hbm, ua_hbm, outb)
            else:
                run(ub_hbm, ub_hbm, outb)

    return k(rc4, ewsp, ta, tb)


BLK = 1000


def _tc_pre_body(degt_ref, x_ref, b_ref, bs_ref, u1_ref, dinv_ref, scale_ref):
    i = pl.program_id(0)
    deg = 1.0 + jnp.sum(degt_ref[...], axis=1, keepdims=True)
    dinv = lax.rsqrt(deg)
    dinv_ref[...] = dinv
    u1_ref[...] = dinv * x_ref[...]

    @pl.when(i == 0)
    def _():
        lane = lax.broadcasted_iota(jnp.int32, (1, N), 1)
        neq = (b_ref[...] != bs_ref[...]) & (lane > 0)
        nd = jnp.sum(neq.astype(jnp.float32)) + 1.0
        scale_ref[...] = jnp.full((1, 1), 1.0 / G, jnp.float32) * nd


def _tc_pre(degt, x, b2d, bs2d):
    return pl.pallas_call(
        _tc_pre_body,
        grid=(N // BLK,),
        in_specs=[
            pl.BlockSpec((BLK, 2), lambda i: (i, 0)),
            pl.BlockSpec((BLK, IN_CH), lambda i: (i, 0)),
            pl.BlockSpec((1, N), lambda i: (0, 0)),
            pl.BlockSpec((1, N), lambda i: (0, 0)),
        ],
        out_specs=[
            pl.BlockSpec((BLK, IN_CH), lambda i: (i, 0)),
            pl.BlockSpec((BLK, 1), lambda i: (i, 0)),
            pl.BlockSpec((1, 1), lambda i: (0, 0)),
        ],
        out_shape=[
            jax.ShapeDtypeStruct((N, IN_CH), jnp.float32),
            jax.ShapeDtypeStruct((N, 1), jnp.float32),
            jax.ShapeDtypeStruct((1, 1), jnp.float32),
        ],
    )(degt, x, b2d, bs2d)


def _tc_layer_body(mode, emit_u, hprev_ref, ta_ref, tb_ref, dinv_ref,
                   wt_ref, b_ref, h_ref, ua_ref, ub_ref):
    dinv = dinv_ref[...]
    if mode == "sum":
        t = ta_ref[...] + tb_ref[...]
    else:
        t = jnp.concatenate([ta_ref[:, :CH], tb_ref[:, :CH]], axis=1)
    z = ALPHA * hprev_ref[...] + BETA * (dinv * t)
    h = jnp.tanh(
        jnp.dot(z, wt_ref[...], preferred_element_type=jnp.float32)
        + b_ref[...])
    h_ref[...] = h
    if emit_u:
        u = dinv * h
        zp = jnp.zeros((BLK, IN_CH - CH), jnp.float32)
        ua_ref[...] = jnp.concatenate([u[:, :CH], zp], axis=1)
        ub_ref[...] = jnp.concatenate([u[:, CH:], zp], axis=1)


def _tc_layer(mode, hprev, ta, tb, dinv, wt, b, cin, emit_u=True):
    return pl.pallas_call(
        functools.partial(_tc_layer_body, mode, emit_u),
        grid=(N // BLK,),
        in_specs=[
            pl.BlockSpec((BLK, cin), lambda i: (i, 0)),
            pl.BlockSpec((BLK, IN_CH), lambda i: (i, 0)),
            pl.BlockSpec((BLK, IN_CH), lambda i: (i, 0)),
            pl.BlockSpec((BLK, 1), lambda i: (i, 0)),
            pl.BlockSpec((cin, HIDP), lambda i: (0, 0)),
            pl.BlockSpec((1, HIDP), lambda i: (0, 0)),
        ],
        out_specs=[
            pl.BlockSpec((BLK, HIDP), lambda i: (i, 0)),
            pl.BlockSpec((BLK, IN_CH), lambda i: (i, 0)),
            pl.BlockSpec((BLK, IN_CH), lambda i: (i, 0)),
        ],
        out_shape=[
            jax.ShapeDtypeStruct((N, HIDP), jnp.float32),
            jax.ShapeDtypeStruct((N, IN_CH), jnp.float32),
            jax.ShapeDtypeStruct((N, IN_CH), jnp.float32),
        ],
    )(hprev, ta, tb, dinv, wt, b)


FDIM = 16 * HIDP


def _tc_head_body(hg_ref, scale_ref, w1_ref, b1_ref, w2_ref, b2_ref, o_ref):
    hs = hg_ref[...] * scale_ref[0, 0]
    z = jnp.tanh(
        jnp.dot(hs, w1_ref[...], preferred_element_type=jnp.float32)
        + b1_ref[...])
    o_ref[...] = (
        jnp.dot(z, w2_ref[...], preferred_element_type=jnp.float32)
        + b2_ref[...])


def _tc_head(hg, scale, wbig, bfc1, wfc2t, bfc2):
    return pl.pallas_call(
        _tc_head_body,
        grid=(1,),
        in_specs=[
            pl.BlockSpec((G, FDIM), lambda i: (0, 0)),
            pl.BlockSpec((1, 1), lambda i: (0, 0)),
            pl.BlockSpec((FDIM, 2752 // 2), lambda i: (0, 0)),
            pl.BlockSpec((1, 2752 // 2), lambda i: (0, 0)),
            pl.BlockSpec((2752 // 2, OUT_CH), lambda i: (0, 0)),
            pl.BlockSpec((1, OUT_CH), lambda i: (0, 0)),
        ],
        out_specs=pl.BlockSpec((G, OUT_CH), lambda i: (0, 0)),
        out_shape=jax.ShapeDtypeStruct((G, OUT_CH), jnp.float32),
    )(hg, scale, wbig, bfc1, wfc2t, bfc2)


def kernel(x, edge_index, edge_weight, batch,
           W1, b1, Wm0, bm0, Wm1, bm1, Wfc1, bfc1, Wfc2, bfc2):
    rc4 = jnp.stack([edge_index[0].reshape(16, NBT, BB),
                     edge_index[1].reshape(16, NBT, BB)], axis=2)
    ewsp = (edge_weight[:, None] * jnp.ones((1, 16), jnp.float32))
    ewsp = ewsp.reshape(16 * NBT, BB, 16)
    zeros128 = jnp.zeros((N, IN_CH), jnp.float32)
    b2d = batch.reshape(1, N)
    bs2d = jnp.roll(batch, 1).reshape(1, N)

    pad = HIDP - HID
    w1t = jnp.pad(W1.T, ((0, 0), (0, pad)))
    b1p = jnp.pad(b1, (0, pad)).reshape(1, HIDP)
    wm0t = jnp.pad(Wm0.T, ((0, pad), (0, pad)))
    bm0p = jnp.pad(bm0, (0, pad)).reshape(1, HIDP)
    wm1t = jnp.pad(Wm1.T, ((0, pad), (0, pad)))
    bm1p = jnp.pad(bm1, (0, pad)).reshape(1, HIDP)
    wbig = jnp.pad(Wfc1.reshape(2752 // 2, 16, HID),
                   ((0, 0), (0, 0), (0, pad)))
    wbig = wbig.reshape(2752 // 2, FDIM).T
    bfc1p = bfc1.reshape(1, 2752 // 2)
    wfc2t = Wfc2.T
    bfc2p = bfc2.reshape(1, OUT_CH)

    degp = _deg_call(rc4, ewsp, zeros128)
    degt = jnp.swapaxes(degp[:, :, 0], 0, 1)
    u1, dinv, scale = _tc_pre(degt, x, b2d, bs2d)

    p0, p1 = _prop_call(rc4, ewsp, u1, zeros128, edge_split=True)
    h1, u2a, u2b = _tc_layer("sum", x, p0, p1, dinv, w1t, b1p, IN_CH)
    t2a, t2b = _prop_call(rc4, ewsp, u2a, u2b, edge_split=False)
    h2, u3a, u3b = _tc_layer("cat", h1, t2a, t2b, dinv, wm0t, bm0p, HIDP)
    t3a, t3b = _prop_call(rc4, ewsp, u3a, u3b, edge_split=False)
    h3, _, _ = _tc_layer("cat", h2, t3a, t3b, dinv, wm1t, bm1p, HIDP,
                         emit_u=False)

    hg = h3.reshape(G, FDIM)
    return _tc_head(hg, scale, wbig, bfc1p, wfc2t, bfc2p)


# speedup vs baseline: 1.0648x; 1.0648x over previous
import functools

import jax
import jax.numpy as jnp
from jax import lax
from jax.experimental import pallas as pl
from jax.experimental.pallas import tpu as pltpu
from jax.experimental.pallas import tpu_sc as plsc

ALPHA = 0.3
BETA = 1.0 - ALPHA
N = 10000
E = 320000
G = 625
IN_CH = 128
HID = 172
HIDP = 192
CH = HIDP // 2
OUT_CH = 10

BB = 20
NBT = E // (16 * BB)
RCHUNK = 632


def _sc_mesh():
    return plsc.VectorSubcoreMesh(core_axis_name="c", subcore_axis_name="s")


def _part_copy(sid, src, dst):
    r0 = pl.multiple_of(sid * RCHUNK, 8)
    tail = N - 15 * RCHUNK

    @pl.when(sid < 15)
    def _():
        pltpu.sync_copy(src.at[pl.ds(r0, RCHUNK)], dst.at[pl.ds(r0, RCHUNK)])

    @pl.when(sid == 15)
    def _():
        pltpu.sync_copy(src.at[pl.ds(15 * RCHUNK, tail)],
                        dst.at[pl.ds(15 * RCHUNK, tail)])


def _deg_call(rc4, ewsp, zeros128):
    @functools.partial(
        pl.kernel,
        mesh=_sc_mesh(),
        out_type=jax.ShapeDtypeStruct((2, N, IN_CH), jnp.float32),
        scratch_types=[
            pltpu.VMEM((2, BB), jnp.int32),
            pltpu.VMEM((2, BB), jnp.int32),
            pltpu.VMEM((BB,), jnp.float32),
            pltpu.VMEM((BB,), jnp.float32),
            pltpu.VMEM((BB, IN_CH), jnp.float32),
            pltpu.VMEM((BB, IN_CH), jnp.float32),
            pltpu.VMEM_SHARED((N, IN_CH), jnp.float32),
            pltpu.SemaphoreType.DMA,
            pltpu.SemaphoreType.DMA,
            pltpu.SemaphoreType.DMA,
            pltpu.SemaphoreType.DMA,
            pltpu.SemaphoreType.DMA,
            pltpu.SemaphoreType.DMA,
        ],
    )
    def k(rc_hbm, ewsp_hbm, z_hbm, out_hbm,
          rcA, rcB, wspA, wspB, stA, stB, acc, e0, e1, w0, w1, s0, s1):
        cid = lax.axis_index("c")
        sid = lax.axis_index("s")
        nj = NBT // 2

        def zb(b, _):
            for kk in range(IN_CH // 16):
                stA[b, pl.ds(kk * 16, 16)] = jnp.zeros((16,), jnp.float32)
                stB[b, pl.ds(kk * 16, 16)] = jnp.zeros((16,), jnp.float32)
            return 0

        lax.fori_loop(0, BB, zb, 0)
        _part_copy(sid, z_hbm, acc)
        plsc.subcore_barrier()

        def body(jj, _):
            j0 = jj * 2
            jb0 = cid * nj + j0
            jb1 = jb0 + 1
            hr0 = pltpu.make_async_copy(rc_hbm.at[sid, jb0], rcA, e0)
            hw0 = pltpu.make_async_copy(ewsp_hbm.at[sid, jb0], wspA, w0)
            hr1 = pltpu.make_async_copy(rc_hbm.at[sid, jb1], rcB, e1)
            hw1 = pltpu.make_async_copy(ewsp_hbm.at[sid, jb1], wspB, w1)
            hr0.start()
            hw0.start()
            hr1.start()
            hw1.start()
            hr0.wait()
            hw0.wait()

            vA0 = wspA[pl.ds(0, 16)]
            vA1 = wspA[pl.ds(BB - 16, 16)]
            ones16 = jnp.ones((16,), jnp.float32)
            for b in range(BB):
                w = vA0[b] if b < 16 else vA1[b - (BB - 16)]
                stA[b, pl.ds(0, 16)] = w * ones16
            s0c = pltpu.make_async_copy(stA, acc.at[rcA.at[1]], s0)
            s0c.start(add=True)
            hr1.wait()
            hw1.wait()

            vB0 = wspB[pl.ds(0, 16)]
            vB1 = wspB[pl.ds(BB - 16, 16)]
            for b in range(BB):
                w = vB0[b] if b < 16 else vB1[b - (BB - 16)]
                stB[b, pl.ds(0, 16)] = w * ones16
            s1c = pltpu.make_async_copy(stB, acc.at[rcB.at[1]], s1)
            s1c.start(add=True)
            s0c.wait()
            s1c.wait()
            return 0

        lax.fori_loop(0, nj // 2, body, 0)
        plsc.subcore_barrier()
        _part_copy(sid, acc, out_hbm.at[cid])

    return k(rc4, ewsp, zeros128)


def _prop_call(rc4, ewsp, ta, tb, edge_split):
    nj = NBT // 2 if edge_split else NBT

    @functools.partial(
        pl.kernel,
        mesh=_sc_mesh(),
        out_type=[jax.ShapeDtypeStruct((N, IN_CH), jnp.float32)] * 2,
        scratch_types=[
            pltpu.VMEM((2, BB), jnp.int32),
            pltpu.VMEM((2, BB), jnp.int32),
            pltpu.VMEM((2, BB), jnp.int32),
            pltpu.VMEM((2, BB), jnp.int32),
            pltpu.VMEM((BB,), jnp.float32),
            pltpu.VMEM((BB,), jnp.float32),
            pltpu.VMEM((BB, IN_CH), jnp.float32),
            pltpu.VMEM((BB, IN_CH), jnp.float32),
            pltpu.VMEM_SHARED((N, IN_CH), jnp.float32),
        ] + [pltpu.SemaphoreType.DMA] * 10,
    )
    def k(rc_hbm, ewsp_hbm, ua_hbm, ub_hbm, outa, outb,
          rcA, rcB, rcC, rcD, wspA, wspB, rowsA, rowsB, acc,
          erA, erB, erC, erD, ewA, ewB, g0, g1, s0, s1):
        cid = lax.axis_index("c")
        sid = lax.axis_index("s")

        def run(init_hbm, u_hbm, out_hbm):
            _part_copy(sid, init_hbm, acc)
            plsc.subcore_barrier()

            def jb_of(j):
                return cid * nj + j if edge_split else j

            def scale(rows, wsp):
                v0 = wsp[pl.ds(0, 16)]
                v1 = wsp[pl.ds(BB - 16, 16)]
                for b in range(BB):
                    w = v0[b] if b < 16 else v1[b - (BB - 16)]
                    for kk in range(IN_CH // 16):
                        sl2 = pl.ds(kk * 16, 16)
                        rows[b, sl2] = rows[b, sl2] * w

            mac = pltpu.make_async_copy

            def body(jj, _):
                j = jj * 4
                jb0, jb1, jb2, jb3 = (jb_of(j), jb_of(j + 1),
                                      jb_of(j + 2), jb_of(j + 3))
                hrA = mac(rc_hbm.at[sid, jb0], rcA, erA)
                hrB = mac(rc_hbm.at[sid, jb1], rcB, erB)
                hrC = mac(rc_hbm.at[sid, jb2], rcC, erC)
                hrD = mac(rc_hbm.at[sid, jb3], rcD, erD)
                hwA = mac(ewsp_hbm.at[sid, jb0], wspA, ewA)
                hwB = mac(ewsp_hbm.at[sid, jb1], wspB, ewB)
                hrA.start()
                hrB.start()
                hrC.start()
                hrD.start()
                hwA.start()
                hwB.start()
                hrA.wait()
                gA = mac(u_hbm.at[rcA.at[0]], rowsA, g0)
                gA.start()
                hrB.wait()
                gB = mac(u_hbm.at[rcB.at[0]], rowsB, g1)
                gB.start()
                gA.wait()
                hwA.wait()
                scale(rowsA, wspA)
                sA = mac(rowsA, acc.at[rcA.at[1]], s0)
                sA.start(add=True)
                hwA2 = mac(ewsp_hbm.at[sid, jb2], wspA, ewA)
                hwA2.start()
                gB.wait()
                hwB.wait()
                scale(rowsB, wspB)
                sB = mac(rowsB, acc.at[rcB.at[1]], s1)
                sB.start(add=True)
                hwB2 = mac(ewsp_hbm.at[sid, jb3], wspB, ewB)
                hwB2.start()
                sA.wait()
                hrC.wait()
                gC = mac(u_hbm.at[rcC.at[0]], rowsA, g0)
                gC.start()
                sB.wait()
                hrD.wait()
                gD = mac(u_hbm.at[rcD.at[0]], rowsB, g1)
                gD.start()
                gC.wait()
                hwA2.wait()
                scale(rowsA, wspA)
                sC = mac(rowsA, acc.at[rcC.at[1]], s0)
                sC.start(add=True)
                gD.wait()
                hwB2.wait()
                scale(rowsB, wspB)
                sD = mac(rowsB, acc.at[rcD.at[1]], s1)
                sD.start(add=True)
                sC.wait()
                sD.wait()
                return 0

            lax.fori_loop(0, nj // 4, body, 0)
            plsc.subcore_barrier()
            _part_copy(sid, acc, out_hbm)
            plsc.subcore_barrier()

        @pl.when(cid == 0)
        def _():
            run(ua_hbm, ua_hbm, outa)

        @pl.when(cid != 0)
        def _():
            if edge_split:
                run(ub_hbm, ua_hbm, outb)
            else:
                run(ub_hbm, ub_hbm, outb)

    return k(rc4, ewsp, ta, tb)


BLK = 1000


def _tc_pre_body(degt_ref, x_ref, b_ref, bs_ref, u1_ref, dinv_ref, scale_ref):
    i = pl.program_id(0)
    deg = 1.0 + jnp.sum(degt_ref[...], axis=1, keepdims=True)
    dinv = lax.rsqrt(deg)
    dinv_ref[...] = dinv
    u1_ref[...] = dinv * x_ref[...]

    @pl.when(i == 0)
    def _():
        lane = lax.broadcasted_iota(jnp.int32, (1, N), 1)
        neq = (b_ref[...] != bs_ref[...]) & (lane > 0)
        nd = jnp.sum(neq.astype(jnp.float32)) + 1.0
        scale_ref[...] = jnp.full((1, 1), 1.0 / G, jnp.float32) * nd


def _tc_pre(degt, x, b2d, bs2d):
    return pl.pallas_call(
        _tc_pre_body,
        grid=(N // BLK,),
        in_specs=[
            pl.BlockSpec((BLK, 2), lambda i: (i, 0)),
            pl.BlockSpec((BLK, IN_CH), lambda i: (i, 0)),
            pl.BlockSpec((1, N), lambda i: (0, 0)),
            pl.BlockSpec((1, N), lambda i: (0, 0)),
        ],
        out_specs=[
            pl.BlockSpec((BLK, IN_CH), lambda i: (i, 0)),
            pl.BlockSpec((BLK, 1), lambda i: (i, 0)),
            pl.BlockSpec((1, 1), lambda i: (0, 0)),
        ],
        out_shape=[
            jax.ShapeDtypeStruct((N, IN_CH), jnp.float32),
            jax.ShapeDtypeStruct((N, 1), jnp.float32),
            jax.ShapeDtypeStruct((1, 1), jnp.float32),
        ],
    )(degt, x, b2d, bs2d)


def _tc_layer_body(mode, emit_u, hprev_ref, ta_ref, tb_ref, dinv_ref,
                   wt_ref, b_ref, h_ref, ua_ref, ub_ref):
    dinv = dinv_ref[...]
    if mode == "sum":
        t = ta_ref[...] + tb_ref[...]
    else:
        t = jnp.concatenate([ta_ref[:, :CH], tb_ref[:, :CH]], axis=1)
    z = ALPHA * hprev_ref[...] + BETA * (dinv * t)
    h = jnp.tanh(
        jnp.dot(z, wt_ref[...], preferred_element_type=jnp.float32)
        + b_ref[...])
    h_ref[...] = h
    if emit_u:
        u = dinv * h
        zp = jnp.zeros((BLK, IN_CH - CH), jnp.float32)
        ua_ref[...] = jnp.concatenate([u[:, :CH], zp], axis=1)
        ub_ref[...] = jnp.concatenate([u[:, CH:], zp], axis=1)


def _tc_layer(mode, hprev, ta, tb, dinv, wt, b, cin, emit_u=True):
    return pl.pallas_call(
        functools.partial(_tc_layer_body, mode, emit_u),
        grid=(N // BLK,),
        in_specs=[
            pl.BlockSpec((BLK, cin), lambda i: (i, 0)),
            pl.BlockSpec((BLK, IN_CH), lambda i: (i, 0)),
            pl.BlockSpec((BLK, IN_CH), lambda i: (i, 0)),
            pl.BlockSpec((BLK, 1), lambda i: (i, 0)),
            pl.BlockSpec((cin, HIDP), lambda i: (0, 0)),
            pl.BlockSpec((1, HIDP), lambda i: (0, 0)),
        ],
        out_specs=[
            pl.BlockSpec((BLK, HIDP), lambda i: (i, 0)),
            pl.BlockSpec((BLK, IN_CH), lambda i: (i, 0)),
            pl.BlockSpec((BLK, IN_CH), lambda i: (i, 0)),
        ],
        out_shape=[
            jax.ShapeDtypeStruct((N, HIDP), jnp.float32),
            jax.ShapeDtypeStruct((N, IN_CH), jnp.float32),
            jax.ShapeDtypeStruct((N, IN_CH), jnp.float32),
        ],
    )(hprev, ta, tb, dinv, wt, b)


FDIM = 16 * HIDP


def _tc_head_body(hg_ref, scale_ref, w1_ref, b1_ref, w2_ref, b2_ref, o_ref):
    hs = hg_ref[...] * scale_ref[0, 0]
    z = jnp.tanh(
        jnp.dot(hs, w1_ref[...], preferred_element_type=jnp.float32)
        + b1_ref[...])
    o_ref[...] = (
        jnp.dot(z, w2_ref[...], preferred_element_type=jnp.float32)
        + b2_ref[...])


def _tc_head(hg, scale, wbig, bfc1, wfc2t, bfc2):
    return pl.pallas_call(
        _tc_head_body,
        grid=(1,),
        in_specs=[
            pl.BlockSpec((G, FDIM), lambda i: (0, 0)),
            pl.BlockSpec((1, 1), lambda i: (0, 0)),
            pl.BlockSpec((FDIM, 2752 // 2), lambda i: (0, 0)),
            pl.BlockSpec((1, 2752 // 2), lambda i: (0, 0)),
            pl.BlockSpec((2752 // 2, OUT_CH), lambda i: (0, 0)),
            pl.BlockSpec((1, OUT_CH), lambda i: (0, 0)),
        ],
        out_specs=pl.BlockSpec((G, OUT_CH), lambda i: (0, 0)),
        out_shape=jax.ShapeDtypeStruct((G, OUT_CH), jnp.float32),
    )(hg, scale, wbig, bfc1, wfc2t, bfc2)


def kernel(x, edge_index, edge_weight, batch,
           W1, b1, Wm0, bm0, Wm1, bm1, Wfc1, bfc1, Wfc2, bfc2):
    rc4 = jnp.stack([edge_index[0].reshape(16, NBT, BB),
                     edge_index[1].reshape(16, NBT, BB)], axis=2)
    ew3 = edge_weight.reshape(16, NBT, BB)
    zeros128 = jnp.zeros((N, IN_CH), jnp.float32)
    b2d = batch.reshape(1, N)
    bs2d = jnp.roll(batch, 1).reshape(1, N)

    pad = HIDP - HID
    w1t = jnp.pad(W1.T, ((0, 0), (0, pad)))
    b1p = jnp.pad(b1, (0, pad)).reshape(1, HIDP)
    wm0t = jnp.pad(Wm0.T, ((0, pad), (0, pad)))
    bm0p = jnp.pad(bm0, (0, pad)).reshape(1, HIDP)
    wm1t = jnp.pad(Wm1.T, ((0, pad), (0, pad)))
    bm1p = jnp.pad(bm1, (0, pad)).reshape(1, HIDP)
    wbig = jnp.pad(Wfc1.reshape(2752 // 2, 16, HID),
                   ((0, 0), (0, 0), (0, pad)))
    wbig = wbig.reshape(2752 // 2, FDIM).T
    bfc1p = bfc1.reshape(1, 2752 // 2)
    wfc2t = Wfc2.T
    bfc2p = bfc2.reshape(1, OUT_CH)

    degp = _deg_call(rc4, ew3, zeros128)
    degt = jnp.swapaxes(degp[:, :, 0], 0, 1)
    u1, dinv, scale = _tc_pre(degt, x, b2d, bs2d)

    p0, p1 = _prop_call(rc4, ew3, u1, zeros128, edge_split=True)
    h1, u2a, u2b = _tc_layer("sum", x, p0, p1, dinv, w1t, b1p, IN_CH)
    t2a, t2b = _prop_call(rc4, ew3, u2a, u2b, edge_split=False)
    h2, u3a, u3b = _tc_layer("cat", h1, t2a, t2b, dinv, wm0t, bm0p, HIDP)
    t3a, t3b = _prop_call(rc4, ew3, u3a, u3b, edge_split=False)
    h3, _, _ = _tc_layer("cat", h2, t3a, t3b, dinv, wm1t, bm1p, HIDP,
                         emit_u=False)

    hg = h3.reshape(G, FDIM)
    return _tc_head(hg, scale, wbig, bfc1p, wfc2t, bfc2p)
